# baseline (speedup 1.0000x reference)
import functools

import jax
import jax.numpy as jnp
from jax import lax
from jax.experimental import pallas as pl
from jax.experimental.pallas import tpu as pltpu
from jax.experimental.pallas import tpu_sc as plsc

N = 10000
NP = 10240
D = 128
E = 320000
NC = 2
NS = 16
NW = NC * NS
EPW = E // NW
CK = 128
NCHF = 79
NFULL = 78 * CK
DEGW = 128
DEGC = 8
RPT = NP // NS

_mesh = plsc.VectorSubcoreMesh(
    core_axis_name="c", subcore_axis_name="s", num_cores=NC, num_subcores=NS
)


def _make_deg(w):
    @functools.partial(
        pl.kernel,
        out_type=jax.ShapeDtypeStruct((NC, NP, w), jnp.float32),
        mesh=_mesh,
        scratch_types=[
            pltpu.VMEM((NCHF, CK), jnp.int32),
            pltpu.VMEM((CK, w), jnp.float32),
            pltpu.VMEM_SHARED((NP, w), jnp.float32),
        ],
    )
    def degk(dst_hbm, ones_hbm, zeros_hbm, out_hbm, dst_v, ones_v, acc):
        c = lax.axis_index("c")
        s = lax.axis_index("s")
        wid = c * NS + s
        pltpu.sync_copy(ones_hbm, ones_v)
        pltpu.sync_copy(dst_hbm.at[wid], dst_v)
        pltpu.sync_copy(zeros_hbm, acc.at[pl.ds(s * RPT, RPT)])
        plsc.subcore_barrier()

        def body(j, carry):
            pltpu.sync_copy(ones_v, acc.at[dst_v.at[j]], add=True)
            return carry

        lax.fori_loop(0, NCHF, body, 0)
        plsc.subcore_barrier()
        pltpu.sync_copy(acc.at[pl.ds(s * RPT, RPT)],
                        out_hbm.at[c, pl.ds(s * RPT, RPT)])

    return degk


_deg_kernel = _make_deg(DEGW)


@functools.partial(
    pl.kernel,
    out_type=jax.ShapeDtypeStruct((NC, NP, D), jnp.float32),
    mesh=_mesh,
    scratch_types=[
        pltpu.VMEM((2, CK), jnp.int32),
        pltpu.VMEM((2, CK), jnp.int32),
        pltpu.VMEM((2, CK, D), jnp.float32),
        pltpu.VMEM_SHARED((NP, D), jnp.float32),
        pltpu.SemaphoreType.DMA((2,)),
        pltpu.SemaphoreType.DMA((2,)),
        pltpu.SemaphoreType.DMA((2,)),
    ],
)
def _spmm_kernel(g_hbm, src_hbm, dst_hbm, zeros_hbm, out_hbm,
                 si, di, rows2, acc, gsem, ssem, dsem):
    c = lax.axis_index("c")
    s = lax.axis_index("s")
    wid = c * NS + s
    pltpu.sync_copy(zeros_hbm, acc.at[pl.ds(s * RPT, RPT)])
    plsc.subcore_barrier()

    pltpu.sync_copy(src_hbm.at[wid, 0], si.at[0])
    pltpu.sync_copy(dst_hbm.at[wid, 0], di.at[0])
    pltpu.async_copy(src_hbm.at[wid, 1], si.at[1], ssem.at[1])
    pltpu.async_copy(dst_hbm.at[wid, 1], di.at[1], dsem.at[1])
    pltpu.async_copy(g_hbm.at[si.at[0]], rows2.at[0], gsem.at[0])

    def body(j, carry):
        p = j & 1
        q = 1 - p

        @pl.when(j + 1 < NCHF)
        def _start_next():
            pltpu.make_async_copy(src_hbm.at[wid, j + 1], si.at[q],
                                  ssem.at[q]).wait()
            pltpu.async_copy(g_hbm.at[si.at[q]], rows2.at[q], gsem.at[q])

        pltpu.make_async_copy(g_hbm.at[si.at[p]], rows2.at[p], gsem.at[p]).wait()
        pltpu.sync_copy(rows2.at[p], acc.at[di.at[p]], add=True)

        @pl.when(j + 2 < NCHF)
        def _stage_next_idx():
            pltpu.async_copy(src_hbm.at[wid, j + 2], si.at[p], ssem.at[p])
            pltpu.async_copy(dst_hbm.at[wid, j + 2], di.at[p], dsem.at[p])

        @pl.when(j + 1 < NCHF)
        def _wait_next_dst():
            pltpu.make_async_copy(dst_hbm.at[wid, j + 1], di.at[q],
                                  dsem.at[q]).wait()

        return carry

    lax.fori_loop(0, NCHF, body, 0)
    plsc.subcore_barrier()
    pltpu.sync_copy(acc.at[pl.ds(s * RPT, RPT)], out_hbm.at[c, pl.ds(s * RPT, RPT)])


RB = 1024


def _projscale_body(x_ref, w_ref, b_ref, d_ref, g_ref, n_ref):
    h = lax.dot_general(
        x_ref[...], w_ref[...], (((1,), (1,)), ((), ())),
        preferred_element_type=jnp.float32,
    ) + b_ref[...]
    nrm = lax.rsqrt(1.0 + d_ref[0, :, 0:1] + d_ref[1, :, 0:1])
    n_ref[...] = jnp.broadcast_to(nrm, (RB, DEGC))
    g_ref[...] = h * nrm


_projscale = pl.pallas_call(
    _projscale_body,
    grid=(NP // RB,),
    in_specs=[
        pl.BlockSpec((RB, D), lambda i: (i, 0)),
        pl.BlockSpec((D, D), lambda i: (0, 0)),
        pl.BlockSpec((1, D), lambda i: (0, 0)),
        pl.BlockSpec((NC, RB, DEGW), lambda i: (0, i, 0)),
    ],
    out_specs=[
        pl.BlockSpec((RB, D), lambda i: (i, 0)),
        pl.BlockSpec((RB, DEGC), lambda i: (i, 0)),
    ],
    out_shape=[
        jax.ShapeDtypeStruct((NP, D), jnp.float32),
        jax.ShapeDtypeStruct((NP, DEGC), jnp.float32),
    ],
)


def _merge_body(p_ref, g_ref, n_ref, o_ref, *, npow):
    nrm = n_ref[:, 0:1]
    scale = nrm * nrm if npow == 2 else nrm
    o_ref[...] = (p_ref[0] + p_ref[1] + g_ref[...]) * scale


def _make_merge(npow):
    return pl.pallas_call(
        functools.partial(_merge_body, npow=npow),
        grid=(NP // RB,),
        in_specs=[
            pl.BlockSpec((NC, RB, D), lambda i: (0, i, 0)),
            pl.BlockSpec((RB, D), lambda i: (i, 0)),
            pl.BlockSpec((RB, DEGC), lambda i: (i, 0)),
        ],
        out_specs=pl.BlockSpec((RB, D), lambda i: (i, 0)),
        out_shape=jax.ShapeDtypeStruct((NP, D), jnp.float32),
    )


_merge2 = _make_merge(2)
_merge1 = _make_merge(1)


def kernel(x, edge_index, W, b):
    er = edge_index.astype(jnp.int32).reshape(2, NW, EPW)
    main = er[:, :, :NFULL].reshape(2, NW, NFULL // CK, CK)
    tail = er[:, :, NFULL:]
    pad_src = jnp.zeros((NW, CK - (EPW - NFULL)), jnp.int32)
    pad_dst = jnp.full((NW, CK - (EPW - NFULL)), NP - 1, jnp.int32)
    src = jnp.concatenate(
        [main[0], jnp.concatenate([tail[0], pad_src], 1)[:, None, :]], 1)
    dst = jnp.concatenate(
        [main[1], jnp.concatenate([tail[1], pad_dst], 1)[:, None, :]], 1)
    xp = jnp.pad(x, ((0, NP - N), (0, 0)))
    ones_w = jnp.ones((CK, DEGW), jnp.float32)
    zeros_w = jnp.zeros((RPT, DEGW), jnp.float32)
    zeros128 = jnp.zeros((RPT, D), jnp.float32)

    degp = _deg_kernel(dst, ones_w, zeros_w)
    g1, nrm = _projscale(xp, W, b.reshape(1, D), degp)
    p1 = _spmm_kernel(g1, src, dst, zeros128)
    g2 = _merge2(p1, g1, nrm)
    p2 = _spmm_kernel(g2, src, dst, zeros128)
    h2 = _merge1(p2, g2, nrm)
    return h2[:N]


# speedup vs baseline: 1.6675x; 1.6675x over previous
import functools

import jax
import jax.numpy as jnp
from jax import lax
from jax.experimental import pallas as pl
from jax.experimental.pallas import tpu as pltpu
from jax.experimental.pallas import tpu_sc as plsc

N = 10000
NP = 10240
D = 128
E = 320000
NC = 2
NS = 16
NW = NC * NS
EPW = E // NW
CHUNK = 80
NCH = EPW // CHUNK
DEGW = 128
DEGC = 8
RPT = NP // NS

_mesh = plsc.VectorSubcoreMesh(
    core_axis_name="c", subcore_axis_name="s", num_cores=NC, num_subcores=NS
)


def _make_deg(w):
    @functools.partial(
        pl.kernel,
        out_type=jax.ShapeDtypeStruct((NC, NP, w), jnp.float32),
        mesh=_mesh,
        scratch_types=[
            pltpu.VMEM((NCH, CHUNK), jnp.int32),
            pltpu.VMEM((CHUNK, w), jnp.float32),
            pltpu.VMEM_SHARED((NP, w), jnp.float32),
        ],
    )
    def degk(dst_hbm, ones_hbm, zeros_hbm, out_hbm, dst_v, ones_v, acc):
        c = lax.axis_index("c")
        s = lax.axis_index("s")
        wid = c * NS + s
        pltpu.sync_copy(ones_hbm, ones_v)
        pltpu.sync_copy(dst_hbm.at[wid], dst_v)
        pltpu.sync_copy(zeros_hbm, acc.at[pl.ds(s * RPT, RPT)])
        plsc.subcore_barrier()

        def body(j, carry):
            pltpu.sync_copy(ones_v, acc.at[dst_v.at[j]], add=True)
            return carry

        lax.fori_loop(0, NCH, body, 0)
        plsc.subcore_barrier()
        pltpu.sync_copy(acc.at[pl.ds(s * RPT, RPT)],
                        out_hbm.at[c, pl.ds(s * RPT, RPT)])

    return degk


_deg_kernel = _make_deg(DEGW)


@functools.partial(
    pl.kernel,
    out_type=jax.ShapeDtypeStruct((NC, NP, D), jnp.float32),
    mesh=_mesh,
    scratch_types=[
        pltpu.VMEM((EPW,), jnp.int32),
        pltpu.VMEM((NCH, CHUNK), jnp.int32),
        pltpu.VMEM((2, CHUNK, D), jnp.float32),
        pltpu.VMEM_SHARED((NP, D), jnp.float32),
        pltpu.SemaphoreType.DMA((2,)),
    ],
)
def _spmm_kernel(g_hbm, src_hbm, dst_hbm, zeros_hbm, out_hbm,
                 src_v, dst_v, rows2, acc, sem2):
    c = lax.axis_index("c")
    s = lax.axis_index("s")
    wid = c * NS + s
    pltpu.sync_copy(src_hbm.at[wid], src_v)
    pltpu.sync_copy(dst_hbm.at[wid], dst_v)
    pltpu.sync_copy(zeros_hbm, acc.at[pl.ds(s * RPT, RPT)])
    plsc.subcore_barrier()

    pltpu.async_copy(g_hbm.at[src_v.at[pl.ds(0, CHUNK)]], rows2.at[0], sem2.at[0])

    def body(j, carry):
        p = j & 1

        @pl.when(j + 1 < NCH)
        def _start_next():
            nxt = src_v.at[pl.ds(pl.multiple_of((j + 1) * CHUNK, CHUNK), CHUNK)]
            pltpu.async_copy(g_hbm.at[nxt], rows2.at[1 - p], sem2.at[1 - p])

        cur = src_v.at[pl.ds(pl.multiple_of(j * CHUNK, CHUNK), CHUNK)]
        pltpu.make_async_copy(g_hbm.at[cur], rows2.at[p], sem2.at[p]).wait()
        pltpu.sync_copy(rows2.at[p], acc.at[dst_v.at[j]], add=True)
        return carry

    lax.fori_loop(0, NCH, body, 0)
    plsc.subcore_barrier()
    pltpu.sync_copy(acc.at[pl.ds(s * RPT, RPT)], out_hbm.at[c, pl.ds(s * RPT, RPT)])


RB = 1024


def _projscale_body(x_ref, w_ref, b_ref, d_ref, g_ref, n_ref):
    h = lax.dot_general(
        x_ref[...], w_ref[...], (((1,), (1,)), ((), ())),
        preferred_element_type=jnp.float32,
    ) + b_ref[...]
    nrm = lax.rsqrt(1.0 + d_ref[0, :, 0:1] + d_ref[1, :, 0:1])
    n_ref[...] = jnp.broadcast_to(nrm, (RB, DEGC))
    g_ref[...] = h * nrm


_projscale = pl.pallas_call(
    _projscale_body,
    grid=(NP // RB,),
    in_specs=[
        pl.BlockSpec((RB, D), lambda i: (i, 0)),
        pl.BlockSpec((D, D), lambda i: (0, 0)),
        pl.BlockSpec((1, D), lambda i: (0, 0)),
        pl.BlockSpec((NC, RB, DEGW), lambda i: (0, i, 0)),
    ],
    out_specs=[
        pl.BlockSpec((RB, D), lambda i: (i, 0)),
        pl.BlockSpec((RB, DEGC), lambda i: (i, 0)),
    ],
    out_shape=[
        jax.ShapeDtypeStruct((NP, D), jnp.float32),
        jax.ShapeDtypeStruct((NP, DEGC), jnp.float32),
    ],
)


def _merge_body(p_ref, g_ref, n_ref, o_ref, *, npow):
    nrm = n_ref[:, 0:1]
    scale = nrm * nrm if npow == 2 else nrm
    o_ref[...] = (p_ref[0] + p_ref[1] + g_ref[...]) * scale


def _make_merge(npow):
    return pl.pallas_call(
        functools.partial(_merge_body, npow=npow),
        grid=(NP // RB,),
        in_specs=[
            pl.BlockSpec((NC, RB, D), lambda i: (0, i, 0)),
            pl.BlockSpec((RB, D), lambda i: (i, 0)),
            pl.BlockSpec((RB, DEGC), lambda i: (i, 0)),
        ],
        out_specs=pl.BlockSpec((RB, D), lambda i: (i, 0)),
        out_shape=jax.ShapeDtypeStruct((NP, D), jnp.float32),
    )


_merge2 = _make_merge(2)
_merge1 = _make_merge(1)


def kernel(x, edge_index, W, b):
    src = edge_index[0].astype(jnp.int32).reshape(NW, EPW)
    dst = edge_index[1].astype(jnp.int32).reshape(NW, NCH, CHUNK)
    xp = jnp.pad(x, ((0, NP - N), (0, 0)))
    ones_w = jnp.ones((CHUNK, DEGW), jnp.float32)
    zeros_w = jnp.zeros((RPT, DEGW), jnp.float32)
    zeros128 = jnp.zeros((RPT, D), jnp.float32)

    degp = _deg_kernel(dst, ones_w, zeros_w)
    g1, nrm = _projscale(xp, W, b.reshape(1, D), degp)
    p1 = _spmm_kernel(g1, src, dst, zeros128)
    g2 = _merge2(p1, g1, nrm)
    p2 = _spmm_kernel(g2, src, dst, zeros128)
    h2 = _merge1(p2, g2, nrm)
    return h2[:N]


# speedup vs baseline: 1.6773x; 1.0059x over previous
import functools

import jax
import jax.numpy as jnp
from jax import lax
from jax.experimental import pallas as pl
from jax.experimental.pallas import tpu as pltpu
from jax.experimental.pallas import tpu_sc as plsc

N = 10000
NP = 10240
D = 128
E = 320000
NC = 2
NS = 16
NW = NC * NS
EPW = E // NW
CHUNK = 80
NCH = EPW // CHUNK
DEGW = 128
DEGC = 8
RPT = NP // NS

_mesh = plsc.VectorSubcoreMesh(
    core_axis_name="c", subcore_axis_name="s", num_cores=NC, num_subcores=NS
)


def _make_deg(w):
    @functools.partial(
        pl.kernel,
        out_type=jax.ShapeDtypeStruct((NC, NP, w), jnp.float32),
        mesh=_mesh,
        scratch_types=[
            pltpu.VMEM((NCH, CHUNK), jnp.int32),
            pltpu.VMEM((CHUNK, w), jnp.float32),
            pltpu.VMEM_SHARED((NP, w), jnp.float32),
            pltpu.SemaphoreType.DMA((2,)),
        ],
    )
    def degk(dst_hbm, ones_hbm, zeros_hbm, out_hbm, dst_v, ones_v, acc, scsem):
        c = lax.axis_index("c")
        s = lax.axis_index("s")
        wid = c * NS + s
        pltpu.sync_copy(ones_hbm, ones_v)
        pltpu.sync_copy(dst_hbm.at[wid], dst_v)
        pltpu.sync_copy(zeros_hbm, acc.at[pl.ds(s * RPT, RPT)])
        plsc.subcore_barrier()

        def body(j, carry):
            p = j & 1

            @pl.when(j >= 2)
            def _drain():
                pltpu.make_async_copy(ones_v, acc.at[dst_v.at[j - 2]],
                                      scsem.at[p]).wait()

            pltpu.make_async_copy(ones_v, acc.at[dst_v.at[j]],
                                  scsem.at[p]).start(add=True)
            return carry

        lax.fori_loop(0, NCH, body, 0)
        pltpu.make_async_copy(ones_v, acc.at[dst_v.at[0]], scsem.at[0]).wait()
        pltpu.make_async_copy(ones_v, acc.at[dst_v.at[0]], scsem.at[1]).wait()
        plsc.subcore_barrier()
        pltpu.sync_copy(acc.at[pl.ds(s * RPT, RPT)],
                        out_hbm.at[c, pl.ds(s * RPT, RPT)])

    return degk


_deg_kernel = _make_deg(DEGW)


@functools.partial(
    pl.kernel,
    out_type=jax.ShapeDtypeStruct((NC, NP, D), jnp.float32),
    mesh=_mesh,
    scratch_types=[
        pltpu.VMEM((EPW,), jnp.int32),
        pltpu.VMEM((NCH, CHUNK), jnp.int32),
        pltpu.VMEM((2, CHUNK, D), jnp.float32),
        pltpu.VMEM_SHARED((NP, D), jnp.float32),
        pltpu.SemaphoreType.DMA((2,)),
        pltpu.SemaphoreType.DMA((2,)),
    ],
)
def _spmm_kernel(g_hbm, src_hbm, dst_hbm, zeros_hbm, out_hbm,
                 src_v, dst_v, rows2, acc, sem2, scsem):
    c = lax.axis_index("c")
    s = lax.axis_index("s")
    wid = c * NS + s
    pltpu.sync_copy(src_hbm.at[wid], src_v)
    pltpu.sync_copy(dst_hbm.at[wid], dst_v)
    pltpu.sync_copy(zeros_hbm, acc.at[pl.ds(s * RPT, RPT)])
    plsc.subcore_barrier()

    pltpu.async_copy(g_hbm.at[src_v.at[pl.ds(0, CHUNK)]], rows2.at[0], sem2.at[0])

    def body(j, carry):
        p = j & 1
        q = 1 - p

        @pl.when(j + 1 < NCH)
        def _start_next():
            @pl.when(j >= 1)
            def _drain_prev_scatter():
                pltpu.make_async_copy(rows2.at[q], acc.at[dst_v.at[j - 1]],
                                      scsem.at[q]).wait()

            nxt = src_v.at[pl.ds(pl.multiple_of((j + 1) * CHUNK, CHUNK), CHUNK)]
            pltpu.async_copy(g_hbm.at[nxt], rows2.at[q], sem2.at[q])

        cur = src_v.at[pl.ds(pl.multiple_of(j * CHUNK, CHUNK), CHUNK)]
        pltpu.make_async_copy(g_hbm.at[cur], rows2.at[p], sem2.at[p]).wait()
        pltpu.make_async_copy(rows2.at[p], acc.at[dst_v.at[j]],
                              scsem.at[p]).start(add=True)
        return carry

    lax.fori_loop(0, NCH, body, 0)
    pltpu.make_async_copy(rows2.at[0], acc.at[dst_v.at[NCH - 1]], scsem.at[0]).wait()
    pltpu.make_async_copy(rows2.at[1], acc.at[dst_v.at[NCH - 2]], scsem.at[1]).wait()
    plsc.subcore_barrier()
    pltpu.sync_copy(acc.at[pl.ds(s * RPT, RPT)], out_hbm.at[c, pl.ds(s * RPT, RPT)])


RB = 1024


def _projscale_body(x_ref, w_ref, b_ref, d_ref, g_ref, n_ref):
    h = lax.dot_general(
        x_ref[...], w_ref[...], (((1,), (1,)), ((), ())),
        preferred_element_type=jnp.float32,
    ) + b_ref[...]
    nrm = lax.rsqrt(1.0 + d_ref[0, :, 0:1] + d_ref[1, :, 0:1])
    n_ref[...] = jnp.broadcast_to(nrm, (RB, DEGC))
    g_ref[...] = h * nrm


_projscale = pl.pallas_call(
    _projscale_body,
    grid=(NP // RB,),
    in_specs=[
        pl.BlockSpec((RB, D), lambda i: (i, 0)),
        pl.BlockSpec((D, D), lambda i: (0, 0)),
        pl.BlockSpec((1, D), lambda i: (0, 0)),
        pl.BlockSpec((NC, RB, DEGW), lambda i: (0, i, 0)),
    ],
    out_specs=[
        pl.BlockSpec((RB, D), lambda i: (i, 0)),
        pl.BlockSpec((RB, DEGC), lambda i: (i, 0)),
    ],
    out_shape=[
        jax.ShapeDtypeStruct((NP, D), jnp.float32),
        jax.ShapeDtypeStruct((NP, DEGC), jnp.float32),
    ],
)


def _merge_body(p_ref, g_ref, n_ref, o_ref, *, npow):
    nrm = n_ref[:, 0:1]
    scale = nrm * nrm if npow == 2 else nrm
    o_ref[...] = (p_ref[0] + p_ref[1] + g_ref[...]) * scale


def _make_merge(npow):
    return pl.pallas_call(
        functools.partial(_merge_body, npow=npow),
        grid=(NP // RB,),
        in_specs=[
            pl.BlockSpec((NC, RB, D), lambda i: (0, i, 0)),
            pl.BlockSpec((RB, D), lambda i: (i, 0)),
            pl.BlockSpec((RB, DEGC), lambda i: (i, 0)),
        ],
        out_specs=pl.BlockSpec((RB, D), lambda i: (i, 0)),
        out_shape=jax.ShapeDtypeStruct((NP, D), jnp.float32),
    )


_merge2 = _make_merge(2)
_merge1 = _make_merge(1)


def kernel(x, edge_index, W, b):
    src = edge_index[0].astype(jnp.int32).reshape(NW, EPW)
    dst = edge_index[1].astype(jnp.int32).reshape(NW, NCH, CHUNK)
    xp = jnp.pad(x, ((0, NP - N), (0, 0)))
    ones_w = jnp.ones((CHUNK, DEGW), jnp.float32)
    zeros_w = jnp.zeros((RPT, DEGW), jnp.float32)
    zeros128 = jnp.zeros((RPT, D), jnp.float32)

    degp = _deg_kernel(dst, ones_w, zeros_w)
    g1, nrm = _projscale(xp, W, b.reshape(1, D), degp)
    p1 = _spmm_kernel(g1, src, dst, zeros128)
    g2 = _merge2(p1, g1, nrm)
    p2 = _spmm_kernel(g2, src, dst, zeros128)
    h2 = _merge1(p2, g2, nrm)
    return h2[:N]
